# baseline (speedup 1.0000x reference)
import functools

import jax
import jax.numpy as jnp
from jax import lax
from jax.experimental import pallas as pl
from jax.experimental.pallas import tpu as pltpu
from jax.experimental.pallas import tpu_sc as plsc

N_USER = 4000
N_NODES = 10000
EMBED = 128
NC, NS = 2, 16
NW = NC * NS
CHUNK = 128
NROWS = 10240
ROWS_PER_TILE = NROWS // NS
E_HALF = 160000
HCH = 1280
CPW = HCH // NS
SEGCH = 40
TRASH_ROW = N_NODES
SPM_ROWS = N_NODES + 16

_SC_MESH = dict(core_axis_name="c", subcore_axis_name="s",
                num_cores=NC, num_subcores=NS)


def _sc_pipeline(table, acc, src_v, dst_v, bufs, gsems, ssems, ncw):
    nh = len(bufs) // 2
    A = tuple(range(nh))
    B = tuple(range(nh, 2 * nh))

    def fire_g(b, j):
        pltpu.async_copy(table.at[src_v.at[j]], bufs[b], gsems[b])

    def wait_g(b):
        pltpu.make_async_copy(table.at[src_v.at[0]], bufs[b], gsems[b]).wait()

    def fire_s(b, j):
        pltpu.async_copy(bufs[b], acc.at[dst_v.at[j]], ssems[b], add=True)

    def wait_s(b):
        pltpu.make_async_copy(bufs[b], acc.at[dst_v.at[0]], ssems[b]).wait()

    for i, b in enumerate(A):
        fire_g(b, i)

    def body(g2, carry):
        j0 = 2 * nh * g2
        for i, b in enumerate(B):
            @pl.when(g2 > 0)
            def _(b=b):
                wait_s(b)
            fire_g(b, j0 + nh + i)
        for i, b in enumerate(A):
            wait_g(b)
            fire_s(b, j0 + i)
        for i, b in enumerate(A):
            wait_s(b)

            @pl.when(j0 + 2 * nh + i < ncw)
            def _(b=b, i=i, j0=j0):
                fire_g(b, j0 + 2 * nh + i)
        for i, b in enumerate(B):
            wait_g(b)
            fire_s(b, j0 + nh + i)
        return carry

    lax.fori_loop(0, ncw // (2 * nh), body, 0)
    for b in B:
        wait_s(b)


def _cnt16():
    @functools.partial(
        pl.kernel,
        mesh=plsc.VectorSubcoreMesh(**_SC_MESH),
        compiler_params=pltpu.CompilerParams(use_tc_tiling_on_sc=False),
        out_type=jax.ShapeDtypeStruct((NC, NROWS, 16), jnp.float32),
        scratch_types=[
            pltpu.VMEM((CPW, CHUNK), jnp.int32),
        ] + [pltpu.VMEM((CHUNK, 16), jnp.float32)] * 2 + [
            pltpu.VMEM_SHARED((NROWS, 16), jnp.float32),
        ] + [pltpu.SemaphoreType.DMA] * 2,
    )
    def cnt(dst0_hbm, dst1_hbm, zeros_hbm, out_hbm, dst_v, *rest):
        bufs, acc, ssems = rest[:2], rest[2], rest[3:5]
        cid = lax.axis_index("c")
        sid = lax.axis_index("s")
        rpt = pl.ds(sid * ROWS_PER_TILE, ROWS_PER_TILE)
        pltpu.sync_copy(zeros_hbm, acc.at[rpt])

        @pl.when(cid == 0)
        def _():
            pltpu.sync_copy(dst0_hbm.at[pl.ds(sid * CPW, CPW)], dst_v)

        @pl.when(cid == 1)
        def _():
            pltpu.sync_copy(dst1_hbm.at[pl.ds(sid * CPW, CPW)], dst_v)
        for b in range(2):
            def fill(i, carry, b=b):
                bufs[b][i, :] = jnp.ones((16,), jnp.float32)
                return carry
            lax.fori_loop(0, CHUNK, fill, 0)
        plsc.subcore_barrier()

        def body(j2, carry):
            for b in range(2):
                @pl.when(j2 > 0)
                def _(b=b):
                    pltpu.make_async_copy(
                        bufs[b], acc.at[dst_v.at[0]], ssems[b]).wait()
                pltpu.async_copy(
                    bufs[b], acc.at[dst_v.at[2 * j2 + b]], ssems[b], add=True)
            return carry

        lax.fori_loop(0, CPW // 2, body, 0)
        for b in range(2):
            pltpu.make_async_copy(bufs[b], acc.at[dst_v.at[0]], ssems[b]).wait()
        plsc.subcore_barrier()
        pltpu.sync_copy(acc.at[rpt], out_hbm.at[cid, rpt])

    return cnt


def _seg16():
    @functools.partial(
        pl.kernel,
        mesh=plsc.VectorSubcoreMesh(**_SC_MESH),
        compiler_params=pltpu.CompilerParams(use_tc_tiling_on_sc=False),
        out_type=jax.ShapeDtypeStruct((NC, NROWS, 16), jnp.float32),
        scratch_types=[
            pltpu.VMEM((CPW, CHUNK), jnp.int32),
            pltpu.VMEM((CPW, CHUNK), jnp.int32),
        ] + [pltpu.VMEM((CHUNK, 16), jnp.float32)] * 4 + [
            pltpu.VMEM_SHARED((NROWS, 16), jnp.float32),
            pltpu.VMEM_SHARED((NROWS, 16), jnp.float32),
        ] + [pltpu.SemaphoreType.DMA] * 8,
    )
    def seg(table_hbm, src0_hbm, src1_hbm, dst0_hbm, dst1_hbm, zeros_hbm,
            out_hbm, src_v, dst_v, *rest):
        bufs, acc, tbl, gsems, ssems = rest[:4], rest[4], rest[5], rest[6:10], rest[10:14]
        cid = lax.axis_index("c")
        sid = lax.axis_index("s")
        rpt = pl.ds(sid * ROWS_PER_TILE, ROWS_PER_TILE)
        pltpu.sync_copy(zeros_hbm, acc.at[rpt])
        pltpu.sync_copy(table_hbm.at[rpt], tbl.at[rpt])

        @pl.when(cid == 0)
        def _():
            pltpu.sync_copy(src0_hbm.at[pl.ds(sid * CPW, CPW)], src_v)
            pltpu.sync_copy(dst0_hbm.at[pl.ds(sid * CPW, CPW)], dst_v)

        @pl.when(cid == 1)
        def _():
            pltpu.sync_copy(src1_hbm.at[pl.ds(sid * CPW, CPW)], src_v)
            pltpu.sync_copy(dst1_hbm.at[pl.ds(sid * CPW, CPW)], dst_v)
        plsc.subcore_barrier()
        _sc_pipeline(tbl, acc, src_v, dst_v, bufs, gsems, ssems, CPW)
        plsc.subcore_barrier()
        pltpu.sync_copy(acc.at[pl.ds(sid * ROWS_PER_TILE, ROWS_PER_TILE)],
                        out_hbm.at[cid, pl.ds(sid * ROWS_PER_TILE, ROWS_PER_TILE)])

    return seg


def _segfull():
    @functools.partial(
        pl.kernel,
        mesh=plsc.VectorSubcoreMesh(**_SC_MESH),
        compiler_params=pltpu.CompilerParams(use_tc_tiling_on_sc=False),
        out_type=jax.ShapeDtypeStruct((N_NODES, EMBED), jnp.float32),
        scratch_types=[
            pltpu.VMEM((SEGCH, CHUNK), jnp.int32),
            pltpu.VMEM((SEGCH, CHUNK), jnp.int32),
        ] + [pltpu.VMEM((CHUNK, EMBED), jnp.float32)] * 2 + [
            pltpu.VMEM_SHARED((SPM_ROWS, EMBED), jnp.float32),
        ] + [pltpu.SemaphoreType.DMA] * 4,
    )
    def seg(table_hbm, src0_hbm, src1_hbm, dst0_hbm, dst1_hbm, zeros_hbm,
            out_hbm, src_v, dst_v, *rest):
        bufs, spm, gsems, ssems = rest[:2], rest[2], rest[3:5], rest[5:7]
        cid = lax.axis_index("c")
        sid = lax.axis_index("s")

        @pl.when(cid == 0)
        def _():
            pltpu.sync_copy(table_hbm.at[pl.ds(sid * 250, 250)],
                            spm.at[pl.ds(sid * 250, 250)])
            pltpu.sync_copy(zeros_hbm,
                            spm.at[pl.ds(N_USER + sid * 376, 376)])

        @pl.when(cid == 1)
        def _():
            pltpu.sync_copy(table_hbm.at[pl.ds(N_USER + sid * 375, 375)],
                            spm.at[pl.ds(N_USER + sid * 375, 375)])
            pltpu.sync_copy(zeros_hbm.at[pl.ds(0, 250)],
                            spm.at[pl.ds(sid * 250, 250)])
        plsc.subcore_barrier()

        def seg_body(s, carry):
            base = sid * CPW + s * SEGCH

            @pl.when(cid == 0)
            def _():
                pltpu.sync_copy(src0_hbm.at[pl.ds(base, SEGCH)], src_v)
                pltpu.sync_copy(dst0_hbm.at[pl.ds(base, SEGCH)], dst_v)

            @pl.when(cid == 1)
            def _():
                pltpu.sync_copy(src1_hbm.at[pl.ds(base, SEGCH)], src_v)
                pltpu.sync_copy(dst1_hbm.at[pl.ds(base, SEGCH)], dst_v)
            _sc_pipeline(spm, spm, src_v, dst_v, bufs, gsems, ssems, SEGCH)
            return carry

        lax.fori_loop(0, CPW // SEGCH, seg_body, 0)
        plsc.subcore_barrier()

        @pl.when(cid == 0)
        def _():
            pltpu.sync_copy(spm.at[pl.ds(N_USER + sid * 375, 375)],
                            out_hbm.at[pl.ds(N_USER + sid * 375, 375)])

        @pl.when(cid == 1)
        def _():
            pltpu.sync_copy(spm.at[pl.ds(sid * 250, 250)],
                            out_hbm.at[pl.ds(sid * 250, 250)])

    return seg


_seg16_k = _seg16()
_cnt16_k = _cnt16()
_segfull_k = _segfull()

_B_IDX = 96


def _final_gather(t0, t1, t2, t3, idx2d):
    @functools.partial(
        pl.kernel,
        mesh=plsc.VectorSubcoreMesh(**_SC_MESH),
        out_type=jax.ShapeDtypeStruct((NW * _B_IDX, 4 * EMBED), jnp.float32),
        scratch_types=[
            pltpu.VMEM((_B_IDX,), jnp.int32),
            pltpu.VMEM((_B_IDX, EMBED), jnp.float32),
            pltpu.SemaphoreType.DMA,
        ],
    )
    def gath(tab0, tab1, tab2, tab3, idx_hbm, out_hbm, idx_v, rows_v, sem):
        cid = lax.axis_index("c")
        sid = lax.axis_index("s")
        wid = cid * NS + sid
        pltpu.sync_copy(idx_hbm.at[wid], idx_v)
        for t, tab in enumerate((tab0, tab1, tab2, tab3)):
            pltpu.async_copy(tab.at[idx_v], rows_v, sem).wait()
            pltpu.sync_copy(rows_v,
                            out_hbm.at[pl.ds(wid * _B_IDX, _B_IDX),
                                       pl.ds(t * EMBED, EMBED)])

    return gath(t0, t1, t2, t3, idx2d)


_DBLK = 2000


def _prep_kernel(emd, cnt_parts):
    def body(emd_ref, cnt_ref, r_ref, hs0_ref):
        ind = cnt_ref[0] + cnt_ref[1]
        r = jnp.where(ind > 0, lax.rsqrt(jnp.maximum(ind, 1e-30)), 0.0)
        r_ref[...] = r
        hs0_ref[...] = emd_ref[...] * r[:, :1]

    grid = N_NODES // _DBLK
    return pl.pallas_call(
        body,
        grid=(grid,),
        in_specs=[
            pl.BlockSpec((_DBLK, EMBED), lambda i: (i, 0)),
            pl.BlockSpec((2, _DBLK, 16), lambda i: (0, i, 0)),
        ],
        out_specs=[
            pl.BlockSpec((_DBLK, 16), lambda i: (i, 0)),
            pl.BlockSpec((_DBLK, EMBED), lambda i: (i, 0)),
        ],
        out_shape=[
            jax.ShapeDtypeStruct((NROWS, 16), jnp.float32),
            jax.ShapeDtypeStruct((N_NODES, EMBED), jnp.float32),
        ],
    )(emd, cnt_parts)


def _dense_layer(h, a, r16, csum_parts, W1, b1, W2, b2, need_hs=True):
    def body(h_ref, a_ref, r_ref, cs_ref, w1_ref, b1_ref, w2_ref, b2_ref,
             hn_ref, hs_ref=None):
        h = h_ref[...]
        self_node = jnp.dot(h, w1_ref[...], preferred_element_type=jnp.float32) \
            + b1_ref[...]
        a = a_ref[...]
        r = r_ref[:, :1]
        c = r * (cs_ref[0][:, :1] + cs_ref[1][:, :1])
        t = (a * r) * h
        inter = jnp.dot(t, w2_ref[...], preferred_element_type=jnp.float32)
        pre = self_node + (self_node + b2_ref[...]) * c + inter
        hn = jnp.where(pre >= 0, pre, 0.2 * pre)
        nrm = jnp.sqrt(jnp.sum(hn * hn, axis=1, keepdims=True))
        hn = hn / jnp.maximum(nrm, 1e-12)
        hn_ref[...] = hn
        if need_hs:
            hs_ref[...] = hn * r

    grid = N_NODES // _DBLK
    wspec = pl.BlockSpec((EMBED, EMBED), lambda i: (0, 0))
    bspec = pl.BlockSpec((1, EMBED), lambda i: (0, 0))
    return pl.pallas_call(
        body,
        grid=(grid,),
        in_specs=[
            pl.BlockSpec((_DBLK, EMBED), lambda i: (i, 0)),
            pl.BlockSpec((_DBLK, EMBED), lambda i: (i, 0)),
            pl.BlockSpec((_DBLK, 16), lambda i: (i, 0)),
            pl.BlockSpec((2, _DBLK, 16), lambda i: (0, i, 0)),
            wspec, bspec, wspec, bspec,
        ],
        out_specs=[pl.BlockSpec((_DBLK, EMBED), lambda i: (i, 0))] * (
            2 if need_hs else 1),
        out_shape=[jax.ShapeDtypeStruct((N_NODES, EMBED), jnp.float32)] * (
            2 if need_hs else 1),
    )(h, a, r16, csum_parts, W1, b1, W2, b2)


def kernel(user, pos_item, neg_item, src, dst, emd,
           W1_0, b1_0, W2_0, b2_0,
           W1_1, b1_1, W2_1, b2_1,
           W1_2, b1_2, W2_2, b2_2):
    params = [(W1_0, b1_0, W2_0, b2_0),
              (W1_1, b1_1, W2_1, b2_1),
              (W1_2, b1_2, W2_2, b2_2)]

    hpad = HCH * CHUNK - E_HALF
    si = src.astype(jnp.int32)
    di = dst.astype(jnp.int32)
    p0 = jnp.zeros((hpad,), jnp.int32)
    p1 = jnp.full((hpad,), N_USER, jnp.int32)
    pt = jnp.full((hpad,), TRASH_ROW, jnp.int32)
    src0 = jnp.concatenate([si[:E_HALF], p0]).reshape(HCH, CHUNK)
    src1 = jnp.concatenate([si[E_HALF:], p1]).reshape(HCH, CHUNK)
    dst0 = jnp.concatenate([di[:E_HALF], pt]).reshape(HCH, CHUNK)
    dst1 = jnp.concatenate([di[E_HALF:], pt]).reshape(HCH, CHUNK)

    z16 = jnp.zeros((ROWS_PER_TILE, 16), jnp.float32)
    z128 = jnp.zeros((376, EMBED), jnp.float32)

    cnt_parts = _cnt16_k(dst0, dst1, z16)
    r16, hs = _prep_kernel(emd, cnt_parts)
    csum_parts = _seg16_k(r16, src0, src1, dst0, dst1, z16)

    h = emd
    h_tables = []
    for li, (W1, b1, W2, b2) in enumerate(params):
        a = _segfull_k(hs, src0, src1, dst0, dst1, z128)
        out = _dense_layer(h, a, r16, csum_parts, W1, b1, W2, b2,
                           need_hs=(li < 2))
        h = out[0]
        hs = out[1] if li < 2 else None
        h_tables.append(h)

    idx = jnp.concatenate([user.astype(jnp.int32),
                           N_USER + pos_item.astype(jnp.int32),
                           N_USER + neg_item.astype(jnp.int32)]).reshape(NW, _B_IDX)
    res = _final_gather(emd, h_tables[0], h_tables[1], h_tables[2], idx)
    return (res[0:1024], res[1024:2048], res[2048:3072])
